# Initial kernel scaffold; baseline (speedup 1.0000x reference)
import functools
import math

import jax
import jax.numpy as jnp
from jax.experimental import pallas as pl
from jax.experimental.pallas import tpu as pltpu

N = 2048
H = 768
F = 3072
E = 8
FB = 768
NFB = F // FB
EPS = 1e-12


def _moe_body(x_ref, rw_ref, rb_ref, w1_ref, b1_ref, w2_ref, b2_ref,
              lng_ref, lnb_ref, out_ref, acc_ref, comb_ref):
    e = pl.program_id(0)
    fb = pl.program_id(1)

    @pl.when(jnp.logical_and(e == 0, fb == 0))
    def _route():
        x = x_ref[...]
        lg = jax.lax.dot_general(x, rw_ref[...], (((1,), (1,)), ((), ())),
                                 preferred_element_type=jnp.float32)
        lg = lg + rb_ref[...]
        iota = jax.lax.broadcasted_iota(jnp.int32, (N, E), 1)
        v0 = jnp.max(lg, axis=1, keepdims=True)
        i0 = jnp.min(jnp.where(lg == v0, iota, E), axis=1, keepdims=True)
        m0 = iota == i0
        lgm = jnp.where(m0, -jnp.inf, lg)
        v1 = jnp.max(lgm, axis=1, keepdims=True)
        i1 = jnp.min(jnp.where(lgm == v1, iota, E), axis=1, keepdims=True)
        m1 = iota == i1
        ew = jnp.exp(v1 - v0)
        w_hi = 1.0 / (1.0 + ew)
        w_lo = ew / (1.0 + ew)
        comb_ref[...] = jnp.where(m0, w_hi, 0.0) + jnp.where(m1, w_lo, 0.0)
        acc_ref[...] = jnp.zeros_like(acc_ref)

    e_oh = (jax.lax.broadcasted_iota(jnp.int32, (E, 1), 0) == e).astype(jnp.float32)
    comb_col = jnp.dot(comb_ref[...], e_oh, preferred_element_type=jnp.float32)

    x = x_ref[...]
    h = jax.lax.dot_general(x, w1_ref[0], (((1,), (1,)), ((), ())),
                            preferred_element_type=jnp.float32)
    h = h + b1_ref[...]
    h = 0.5 * h * (1.0 + jax.lax.erf(h * (1.0 / math.sqrt(2.0))))
    h = h * comb_col
    acc_ref[...] += jax.lax.dot_general(h, w2_ref[0], (((1,), (1,)), ((), ())),
                                        preferred_element_type=jnp.float32)

    @pl.when(fb == 0)
    def _bias2():
        acc_ref[...] += comb_col * b2_ref[...]

    @pl.when(jnp.logical_and(e == E - 1, fb == NFB - 1))
    def _finish():
        t = acc_ref[...] + x_ref[...]
        mu = jnp.mean(t, axis=1, keepdims=True)
        tc = t - mu
        var = jnp.mean(tc * tc, axis=1, keepdims=True)
        out_ref[...] = tc * jax.lax.rsqrt(var + EPS) * lng_ref[...] + lnb_ref[...]


@jax.jit
def _moe_ffn(flat, router_w, router_b, W1, b1, W2, b2, ln_g, ln_b):
    return pl.pallas_call(
        _moe_body,
        grid=(E, NFB),
        in_specs=[
            pl.BlockSpec((N, H), lambda e, f: (0, 0)),
            pl.BlockSpec((E, H), lambda e, f: (0, 0)),
            pl.BlockSpec((1, E), lambda e, f: (0, 0)),
            pl.BlockSpec((1, FB, H), lambda e, f: (e, f, 0)),
            pl.BlockSpec((1, FB), lambda e, f: (e, f)),
            pl.BlockSpec((1, H, FB), lambda e, f: (e, 0, f)),
            pl.BlockSpec((1, H), lambda e, f: (e, 0)),
            pl.BlockSpec((1, H), lambda e, f: (0, 0)),
            pl.BlockSpec((1, H), lambda e, f: (0, 0)),
        ],
        out_specs=pl.BlockSpec((N, H), lambda e, f: (0, 0)),
        out_shape=jax.ShapeDtypeStruct((N, H), jnp.float32),
        scratch_shapes=[
            pltpu.VMEM((N, H), jnp.float32),
            pltpu.VMEM((N, E), jnp.float32),
        ],
    )(flat, router_w, router_b, W1, b1, W2, b2, ln_g, ln_b)


def kernel(hidden_states, router_w, router_b, W1, b1, W2, b2, ln_g, ln_b):
    bsz, seqlen, h = hidden_states.shape
    flat = hidden_states.reshape(-1, h)
    out = _moe_ffn(flat, router_w, router_b.reshape(1, E), W1, b1, W2, b2,
                   ln_g.reshape(1, h), ln_b.reshape(1, h))
    return out.reshape(bsz, seqlen, h)


# speedup vs baseline: 4.1932x; 4.1932x over previous
import functools
import math

import jax
import jax.numpy as jnp
from jax.experimental import pallas as pl
from jax.experimental.pallas import tpu as pltpu

N = 2048
H = 768
F = 3072
E = 8
FB = 768
NFB = F // FB
EPS = 1e-12


def _moe_body(x_ref, rw_ref, rb_ref, w1_ref, b1_ref, w2_ref, b2_ref,
              lng_ref, lnb_ref, out_ref, acc_ref, comb_ref):
    e = pl.program_id(0)
    fb = pl.program_id(1)

    @pl.when(jnp.logical_and(e == 0, fb == 0))
    def _route():
        x = x_ref[...]
        lg = jax.lax.dot_general(x, rw_ref[...], (((1,), (1,)), ((), ())),
                                 preferred_element_type=jnp.float32)
        lg = lg + rb_ref[...]
        iota = jax.lax.broadcasted_iota(jnp.int32, (N, E), 1)
        v0 = jnp.max(lg, axis=1, keepdims=True)
        i0 = jnp.min(jnp.where(lg == v0, iota, E), axis=1, keepdims=True)
        m0 = iota == i0
        lgm = jnp.where(m0, -jnp.inf, lg)
        v1 = jnp.max(lgm, axis=1, keepdims=True)
        i1 = jnp.min(jnp.where(lgm == v1, iota, E), axis=1, keepdims=True)
        m1 = iota == i1
        ew = jnp.exp(v1 - v0)
        w_hi = 1.0 / (1.0 + ew)
        w_lo = ew / (1.0 + ew)
        comb_ref[...] = jnp.where(m0, w_hi, 0.0) + jnp.where(m1, w_lo, 0.0)
        acc_ref[...] = jnp.zeros_like(acc_ref)

    e_oh = (jax.lax.broadcasted_iota(jnp.int32, (E, 1), 0) == e).astype(jnp.float32)
    comb_col = jnp.dot(comb_ref[...], e_oh, preferred_element_type=jnp.float32)

    x = x_ref[...]
    h = jax.lax.dot_general(x, w1_ref[0], (((1,), (1,)), ((), ())),
                            preferred_element_type=jnp.float32)
    h = h + b1_ref[0]
    h = 0.5 * h * (1.0 + jax.lax.erf(h * (1.0 / math.sqrt(2.0))))
    h = h * comb_col
    acc_ref[...] += jax.lax.dot_general(h, w2_ref[0], (((1,), (1,)), ((), ())),
                                        preferred_element_type=jnp.float32)

    @pl.when(fb == 0)
    def _bias2():
        acc_ref[...] += comb_col * b2_ref[0]

    @pl.when(jnp.logical_and(e == E - 1, fb == NFB - 1))
    def _finish():
        t = acc_ref[...] + x_ref[...]
        mu = jnp.mean(t, axis=1, keepdims=True)
        tc = t - mu
        var = jnp.mean(tc * tc, axis=1, keepdims=True)
        out_ref[...] = tc * jax.lax.rsqrt(var + EPS) * lng_ref[...] + lnb_ref[...]


@jax.jit
def _moe_ffn(flat, router_w, router_b, W1, b1, W2, b2, ln_g, ln_b):
    return pl.pallas_call(
        _moe_body,
        grid=(E, NFB),
        in_specs=[
            pl.BlockSpec((N, H), lambda e, f: (0, 0)),
            pl.BlockSpec((E, H), lambda e, f: (0, 0)),
            pl.BlockSpec((1, E), lambda e, f: (0, 0)),
            pl.BlockSpec((1, FB, H), lambda e, f: (e, f, 0)),
            pl.BlockSpec((1, 1, FB), lambda e, f: (e, 0, f)),
            pl.BlockSpec((1, H, FB), lambda e, f: (e, 0, f)),
            pl.BlockSpec((1, 1, H), lambda e, f: (e, 0, 0)),
            pl.BlockSpec((1, H), lambda e, f: (0, 0)),
            pl.BlockSpec((1, H), lambda e, f: (0, 0)),
        ],
        out_specs=pl.BlockSpec((N, H), lambda e, f: (0, 0)),
        out_shape=jax.ShapeDtypeStruct((N, H), jnp.float32),
        scratch_shapes=[
            pltpu.VMEM((N, H), jnp.float32),
            pltpu.VMEM((N, E), jnp.float32),
        ],
    )(flat, router_w, router_b, W1, b1.reshape(E, 1, F), W2,
      b2.reshape(E, 1, H), ln_g, ln_b)


def kernel(hidden_states, router_w, router_b, W1, b1, W2, b2, ln_g, ln_b):
    bsz, seqlen, h = hidden_states.shape
    flat = hidden_states.reshape(-1, h)
    out = _moe_ffn(flat, router_w, router_b.reshape(1, E), W1, b1, W2, b2,
                   ln_g.reshape(1, h), ln_b.reshape(1, h))
    return out.reshape(bsz, seqlen, h)
